# baseline (speedup 1.0000x reference)
import jax
import jax.numpy as jnp
from jax.experimental import pallas as pl

N = 10000
F = 128
K = 128


def _norm(edge_index, edge_weight, num_nodes, improved):
    fill = 2.0 if improved else 1.0
    row = edge_index[0]
    col = edge_index[1]
    if edge_weight is None:
        edge_weight = jnp.ones((edge_index.shape[1],), dtype=jnp.float32)
    loop = jnp.arange(num_nodes, dtype=edge_index.dtype)
    row = jnp.concatenate([row, loop])
    col = jnp.concatenate([col, loop])
    ew = jnp.concatenate([edge_weight, jnp.full((num_nodes,), fill, dtype=jnp.float32)])
    deg = jnp.zeros((num_nodes,), jnp.float32).at[col].add(ew)
    dinv = jnp.where(deg > 0, 1.0 / jnp.sqrt(deg), 0.0)
    norm = dinv[row] * ew * dinv[col]
    return row, col, norm


def _gcn(x, W, b, edge_index, edge_weight, num_nodes, improved=False):
    row, col, norm = _norm(edge_index, edge_weight, num_nodes, improved)
    h = x @ W
    out = jnp.zeros((num_nodes, h.shape[1]), h.dtype).at[col].add(norm[:, None] * h[row])
    if b is not None:
        out = out + b
    return out


def kernel(x, edge_index, edge_weight, p, W0, W_ih, W_hh, b_ih, b_hh, W1, b1, W2, b2, W3, b3, Wl, bl):
    num_nodes = x.shape[0]
    s = jnp.tanh((x @ p) / jnp.linalg.norm(p))
    vals, perm = jax.lax.top_k(s, K)
    X_t = x[perm] * vals[:, None]
    Gi = X_t @ W_ih.T + b_ih
    Gh = W0 @ W_hh.T + b_hh
    r = jax.nn.sigmoid(Gi[:, :F] + Gh[:, :F])
    z = jax.nn.sigmoid(Gi[:, F:2 * F] + Gh[:, F:2 * F])
    n = jnp.tanh(Gi[:, 2 * F:] + r * Gh[:, 2 * F:])
    Wnew = (1.0 - z) * n + z * W0
    h = _gcn(x, Wnew, None, edge_index, edge_weight, num_nodes, improved=True)
    h = jnp.tanh(_gcn(h, W1, b1, edge_index, None, num_nodes))
    h = jnp.tanh(_gcn(h, W2, b2, edge_index, None, num_nodes))
    h3 = _gcn(h, W3, b3, edge_index, None, num_nodes)
    out = jnp.tanh(h3)
    pred = h3 @ Wl + bl
    return (pred, out)


# speedup vs baseline: 52.8165x; 52.8165x over previous
import functools

import jax
import jax.numpy as jnp
from jax import lax
from jax.experimental import pallas as pl
from jax.experimental.pallas import tpu as pltpu
from jax.experimental.pallas import tpu_sc as plsc

N = 10000
F = 128
E = 320000
K = 128

NC = 2
NS = 16
NW = NC * NS
EPW = E // NW
EB = 2000
NPAD = 10240
STRIPE = NPAD // NS
LANES = 16

_mesh = plsc.VectorSubcoreMesh(
    core_axis_name="c", subcore_axis_name="s", num_cores=NC, num_subcores=NS)


@functools.partial(
    pl.kernel,
    out_type=(
        jax.ShapeDtypeStruct((NC, NPAD), jnp.float32),
        jax.ShapeDtypeStruct((NC, NPAD), jnp.float32),
    ),
    mesh=_mesh,
    compiler_params=pltpu.CompilerParams(needs_layout_passes=False),
    scratch_types=[
        pltpu.VMEM((NPAD,), jnp.float32),
        pltpu.VMEM((NPAD,), jnp.float32),
        pltpu.VMEM((EB,), jnp.int32),
        pltpu.VMEM((EB,), jnp.float32),
        pltpu.VMEM((NS, STRIPE), jnp.float32),
        pltpu.VMEM((STRIPE,), jnp.float32),
        pltpu.VMEM_SHARED((NS, NPAD), jnp.float32),
        pltpu.VMEM_SHARED((NS, NPAD), jnp.float32),
    ],
)
def _deg_kernel(cols_hbm, ew_hbm, degw_hbm, degc_hbm,
                dw, dc, colb, ewb, redin, redout, sh_w, sh_c):
    cid = lax.axis_index("c")
    sid = lax.axis_index("s")
    wid = sid * NC + cid

    zeros16 = jnp.zeros((LANES,), jnp.float32)
    ones16 = jnp.ones((LANES,), jnp.float32)

    def _init(i, _):
        dw[pl.ds(i * LANES, LANES)] = zeros16
        dc[pl.ds(i * LANES, LANES)] = zeros16
        return _
    lax.fori_loop(0, NPAD // LANES, _init, 0)

    base = wid * EPW

    def _block(b, _):
        pltpu.sync_copy(cols_hbm.at[pl.ds(base + b * EB, EB)], colb)
        pltpu.sync_copy(ew_hbm.at[pl.ds(base + b * EB, EB)], ewb)

        def _vec(j, _):
            c16 = colb[pl.ds(j * LANES, LANES)]
            w16 = ewb[pl.ds(j * LANES, LANES)]
            plsc.addupdate_scatter(dw, [c16], w16)
            plsc.addupdate_scatter(dc, [c16], ones16)
            return _
        lax.fori_loop(0, EB // LANES, _vec, 0)
        return _
    lax.fori_loop(0, EPW // EB, _block, 0)

    pltpu.sync_copy(dw, sh_w.at[sid])
    pltpu.sync_copy(dc, sh_c.at[sid])
    plsc.subcore_barrier()

    off = sid * STRIPE
    for sh, out_hbm in ((sh_w, degw_hbm), (sh_c, degc_hbm)):
        pltpu.sync_copy(sh.at[:, pl.ds(off, STRIPE)], redin)

        def _red(i, _):
            acc = zeros16
            for t in range(NS):
                acc = acc + redin[t, pl.ds(i * LANES, LANES)]
            redout[pl.ds(i * LANES, LANES)] = acc
            return _
        lax.fori_loop(0, STRIPE // LANES, _red, 0)
        pltpu.sync_copy(redout, out_hbm.at[cid, pl.ds(off, STRIPE)])


def _degrees_sc(cols, ew):
    dw_parts, dc_parts = _deg_kernel(cols, ew)
    degw = dw_parts.sum(0)[:N]
    degc = dc_parts.sum(0)[:N]
    return degw, degc


def _make_agg(feats, weighted):
    FN = N * feats
    FNPAD = NPAD * feats
    scratch = [
        pltpu.VMEM((FN,), jnp.float32),
        pltpu.VMEM((FNPAD,), jnp.float32),
        pltpu.VMEM((EB,), jnp.int32),
        pltpu.VMEM((EB,), jnp.int32),
        pltpu.VMEM((EB,), jnp.float32),
    ]

    def body(rows_hbm, cols_hbm, ew_hbm, tab_hbm, out_hbm,
             tab, acc, rowb, colb, ewb):
        cid = lax.axis_index("c")
        sid = lax.axis_index("s")
        wid = sid * NC + cid

        zeros16 = jnp.zeros((LANES,), jnp.float32)

        def _init(i, _):
            acc[pl.ds(i * LANES, LANES)] = zeros16
            return _
        lax.fori_loop(0, FNPAD // LANES, _init, 0)
        pltpu.sync_copy(tab_hbm, tab)

        base = wid * EPW

        def _block(b, _):
            pltpu.sync_copy(rows_hbm.at[pl.ds(base + b * EB, EB)], rowb)
            pltpu.sync_copy(cols_hbm.at[pl.ds(base + b * EB, EB)], colb)
            if weighted:
                pltpu.sync_copy(ew_hbm.at[pl.ds(base + b * EB, EB)], ewb)

            def _vec(j, _):
                r16 = rowb[pl.ds(j * LANES, LANES)] * feats
                c16 = colb[pl.ds(j * LANES, LANES)] * feats
                if weighted:
                    w16 = ewb[pl.ds(j * LANES, LANES)]
                for f in range(feats):
                    v = plsc.load_gather(tab, [r16 + f])
                    if weighted:
                        v = v * w16
                    plsc.addupdate_scatter(acc, [c16 + f], v)
                return _
            lax.fori_loop(0, EB // LANES, _vec, 0)
            return _
        lax.fori_loop(0, EPW // EB, _block, 0)

        pltpu.sync_copy(acc, out_hbm.at[wid])

    def body_unweighted(rows_hbm, cols_hbm, tab_hbm, out_hbm,
                        tab, acc, rowb, colb, ewb):
        return body(rows_hbm, cols_hbm, None, tab_hbm, out_hbm,
                    tab, acc, rowb, colb, ewb)

    return functools.partial(
        pl.kernel,
        out_type=jax.ShapeDtypeStruct((NW, FNPAD), jnp.float32),
        mesh=_mesh,
        compiler_params=pltpu.CompilerParams(needs_layout_passes=False),
        scratch_types=scratch,
    )(body if weighted else body_unweighted)


_agg4w = _make_agg(4, True)
_agg4 = _make_agg(4, False)
_agg2 = _make_agg(2, False)


def _agg(fn, feats, row, col, tab, ew=None):
    args = (row, col) + ((ew,) if ew is not None else ()) + (tab.reshape(-1),)
    parts = fn(*args)
    return parts.sum(0)[:N * feats].reshape(N, feats)


def _gcn_agg(h, col, row, norm, num_nodes):
    return jnp.zeros((num_nodes, h.shape[1]), h.dtype).at[col].add(norm[:, None] * h[row])


def kernel(x, edge_index, edge_weight, p, W0, W_ih, W_hh, b_ih, b_hh, W1, b1, W2, b2, W3, b3, Wl, bl):
    num_nodes = x.shape[0]
    row = edge_index[0]
    col = edge_index[1]

    degw_e, degc_e = _degrees_sc(col, edge_weight)
    deg1 = degw_e + 2.0
    deg2 = degc_e + 1.0
    dinv1 = jax.lax.rsqrt(deg1)
    dinv2 = jax.lax.rsqrt(deg2)

    s = jnp.tanh((x @ p) / jnp.linalg.norm(p))
    vals, perm = jax.lax.top_k(s, K)
    X_t = x[perm] * vals[:, None]
    Gi = X_t @ W_ih.T + b_ih
    Gh = W0 @ W_hh.T + b_hh
    r = jax.nn.sigmoid(Gi[:, :F] + Gh[:, :F])
    z = jax.nn.sigmoid(Gi[:, F:2 * F] + Gh[:, F:2 * F])
    n = jnp.tanh(Gi[:, 2 * F:] + r * Gh[:, 2 * F:])
    Wnew = (1.0 - z) * n + z * W0

    d1 = dinv1[:, None]
    d2 = dinv2[:, None]

    xt = (x @ (Wnew @ W1)) * d1
    a1 = d1 * (_agg(_agg4w, 4, row, col, xt, edge_weight) + 2.0 * xt)

    y2 = a1 * d2
    a2 = d2 * (_agg(_agg4, 4, row, col, y2) + y2)
    h2 = jnp.tanh(a2 + b1)

    y3 = (h2 @ W2) * d2
    a3 = d2 * (_agg(_agg4, 4, row, col, y3) + y3)
    h3 = jnp.tanh(a3 + b2)

    y4 = (h3 @ W3) * d2
    a4 = d2 * (_agg(_agg2, 2, row, col, y4) + y4)
    h4 = a4 + b3
    out = jnp.tanh(h4)
    pred = h4 @ Wl + bl
    return (pred, out)


# speedup vs baseline: 60.1348x; 1.1386x over previous
import functools

import jax
import jax.numpy as jnp
from jax import lax
from jax.experimental import pallas as pl
from jax.experimental.pallas import tpu as pltpu
from jax.experimental.pallas import tpu_sc as plsc

N = 10000
F = 128
E = 320000
K = 128

NC = 2
NS = 16
NW = NC * NS
EPW = E // NW
EB = 2000
NPAD = 10240
STRIPE = NPAD // NS
LANES = 16

_mesh = plsc.VectorSubcoreMesh(
    core_axis_name="c", subcore_axis_name="s", num_cores=NC, num_subcores=NS)


@functools.partial(
    pl.kernel,
    out_type=(
        jax.ShapeDtypeStruct((NC, NPAD), jnp.float32),
        jax.ShapeDtypeStruct((NC, NPAD), jnp.float32),
    ),
    mesh=_mesh,
    compiler_params=pltpu.CompilerParams(needs_layout_passes=False),
    scratch_types=[
        pltpu.VMEM((NPAD,), jnp.float32),
        pltpu.VMEM((NPAD,), jnp.float32),
        pltpu.VMEM((EB,), jnp.int32),
        pltpu.VMEM((EB,), jnp.float32),
        pltpu.VMEM((NS, STRIPE), jnp.float32),
        pltpu.VMEM((STRIPE,), jnp.float32),
        pltpu.VMEM_SHARED((NS, NPAD), jnp.float32),
        pltpu.VMEM_SHARED((NS, NPAD), jnp.float32),
    ],
)
def _deg_kernel(cols_hbm, ew_hbm, degw_hbm, degc_hbm,
                dw, dc, colb, ewb, redin, redout, sh_w, sh_c):
    cid = lax.axis_index("c")
    sid = lax.axis_index("s")
    wid = sid * NC + cid

    zeros16 = jnp.zeros((LANES,), jnp.float32)
    ones16 = jnp.ones((LANES,), jnp.float32)

    ZUN = 8

    def _init(i, _):
        for u in range(ZUN):
            dw[pl.ds((i * ZUN + u) * LANES, LANES)] = zeros16
            dc[pl.ds((i * ZUN + u) * LANES, LANES)] = zeros16
        return _
    lax.fori_loop(0, NPAD // (LANES * ZUN), _init, 0)

    base = wid * EPW
    UN = 5

    def _block(b, _):
        pltpu.sync_copy(cols_hbm.at[pl.ds(base + b * EB, EB)], colb)
        pltpu.sync_copy(ew_hbm.at[pl.ds(base + b * EB, EB)], ewb)

        def _vec(j, _):
            for u in range(UN):
                o = (j * UN + u) * LANES
                c16 = colb[pl.ds(o, LANES)]
                w16 = ewb[pl.ds(o, LANES)]
                plsc.addupdate_scatter(dw, [c16], w16)
                plsc.addupdate_scatter(dc, [c16], ones16)
            return _
        lax.fori_loop(0, EB // (LANES * UN), _vec, 0)
        return _
    lax.fori_loop(0, EPW // EB, _block, 0)

    pltpu.sync_copy(dw, sh_w.at[sid])
    pltpu.sync_copy(dc, sh_c.at[sid])
    plsc.subcore_barrier()

    off = sid * STRIPE
    for sh, out_hbm in ((sh_w, degw_hbm), (sh_c, degc_hbm)):
        pltpu.sync_copy(sh.at[:, pl.ds(off, STRIPE)], redin)

        def _red(i, _):
            acc = zeros16
            for t in range(NS):
                acc = acc + redin[t, pl.ds(i * LANES, LANES)]
            redout[pl.ds(i * LANES, LANES)] = acc
            return _
        lax.fori_loop(0, STRIPE // LANES, _red, 0)
        pltpu.sync_copy(redout, out_hbm.at[cid, pl.ds(off, STRIPE)])


def _degrees_sc(cols, ew):
    dw_parts, dc_parts = _deg_kernel(cols, ew)
    degw = dw_parts.sum(0)[:N]
    degc = dc_parts.sum(0)[:N]
    return degw, degc


def _make_agg(feats, weighted):
    FN = N * feats
    FNPAD = NPAD * feats
    scratch = [
        pltpu.VMEM((FN,), jnp.float32),
        pltpu.VMEM((FNPAD,), jnp.float32),
        pltpu.VMEM((EB,), jnp.int32),
        pltpu.VMEM((EB,), jnp.int32),
        pltpu.VMEM((EB,), jnp.float32),
    ]

    def body(rows_hbm, cols_hbm, ew_hbm, tab_hbm, out_hbm,
             tab, acc, rowb, colb, ewb):
        cid = lax.axis_index("c")
        sid = lax.axis_index("s")
        wid = sid * NC + cid

        zeros16 = jnp.zeros((LANES,), jnp.float32)
        ZUN = 8

        def _init(i, _):
            for u in range(ZUN):
                acc[pl.ds((i * ZUN + u) * LANES, LANES)] = zeros16
            return _
        lax.fori_loop(0, FNPAD // (LANES * ZUN), _init, 0)
        pltpu.sync_copy(tab_hbm, tab)

        base = wid * EPW
        UN = 5

        def _block(b, _):
            pltpu.sync_copy(rows_hbm.at[pl.ds(base + b * EB, EB)], rowb)
            pltpu.sync_copy(cols_hbm.at[pl.ds(base + b * EB, EB)], colb)
            if weighted:
                pltpu.sync_copy(ew_hbm.at[pl.ds(base + b * EB, EB)], ewb)

            def _vec(j, _):
                for u in range(UN):
                    o = (j * UN + u) * LANES
                    r16 = rowb[pl.ds(o, LANES)] * feats
                    c16 = colb[pl.ds(o, LANES)] * feats
                    if weighted:
                        w16 = ewb[pl.ds(o, LANES)]
                    for f in range(feats):
                        v = plsc.load_gather(tab, [r16 + f])
                        if weighted:
                            v = v * w16
                        plsc.addupdate_scatter(acc, [c16 + f], v)
                return _
            lax.fori_loop(0, EB // (LANES * UN), _vec, 0)
            return _
        lax.fori_loop(0, EPW // EB, _block, 0)

        pltpu.sync_copy(acc, out_hbm.at[wid])

    def body_unweighted(rows_hbm, cols_hbm, tab_hbm, out_hbm,
                        tab, acc, rowb, colb, ewb):
        return body(rows_hbm, cols_hbm, None, tab_hbm, out_hbm,
                    tab, acc, rowb, colb, ewb)

    return functools.partial(
        pl.kernel,
        out_type=jax.ShapeDtypeStruct((NW, FNPAD), jnp.float32),
        mesh=_mesh,
        compiler_params=pltpu.CompilerParams(needs_layout_passes=False),
        scratch_types=scratch,
    )(body if weighted else body_unweighted)


_agg4w = _make_agg(4, True)
_agg4 = _make_agg(4, False)
_agg2 = _make_agg(2, False)


def _agg(fn, feats, row, col, tab, ew=None):
    args = (row, col) + ((ew,) if ew is not None else ()) + (tab.reshape(-1),)
    parts = fn(*args)
    return parts.sum(0)[:N * feats].reshape(N, feats)


def _gcn_agg(h, col, row, norm, num_nodes):
    return jnp.zeros((num_nodes, h.shape[1]), h.dtype).at[col].add(norm[:, None] * h[row])


def kernel(x, edge_index, edge_weight, p, W0, W_ih, W_hh, b_ih, b_hh, W1, b1, W2, b2, W3, b3, Wl, bl):
    num_nodes = x.shape[0]
    row = edge_index[0]
    col = edge_index[1]

    degw_e, degc_e = _degrees_sc(col, edge_weight)
    deg1 = degw_e + 2.0
    deg2 = degc_e + 1.0
    dinv1 = jax.lax.rsqrt(deg1)
    dinv2 = jax.lax.rsqrt(deg2)

    s = jnp.tanh((x @ p) / jnp.linalg.norm(p))
    vals, perm = jax.lax.top_k(s, K)
    X_t = x[perm] * vals[:, None]
    Gi = X_t @ W_ih.T + b_ih
    Gh = W0 @ W_hh.T + b_hh
    r = jax.nn.sigmoid(Gi[:, :F] + Gh[:, :F])
    z = jax.nn.sigmoid(Gi[:, F:2 * F] + Gh[:, F:2 * F])
    n = jnp.tanh(Gi[:, 2 * F:] + r * Gh[:, 2 * F:])
    Wnew = (1.0 - z) * n + z * W0

    d1 = dinv1[:, None]
    d2 = dinv2[:, None]

    xt = (x @ (Wnew @ W1)) * d1
    a1 = d1 * (_agg(_agg4w, 4, row, col, xt, edge_weight) + 2.0 * xt)

    y2 = a1 * d2
    a2 = d2 * (_agg(_agg4, 4, row, col, y2) + y2)
    h2 = jnp.tanh(a2 + b1)

    y3 = (h2 @ W2) * d2
    a3 = d2 * (_agg(_agg4, 4, row, col, y3) + y3)
    h3 = jnp.tanh(a3 + b2)

    y4 = (h3 @ W3) * d2
    a4 = d2 * (_agg(_agg2, 2, row, col, y4) + y4)
    h4 = a4 + b3
    out = jnp.tanh(h4)
    pred = h4 @ Wl + bl
    return (pred, out)
